# baseline (speedup 1.0000x reference)
import jax
import jax.numpy as jnp
from jax import lax
from jax.experimental import pallas as pl
from jax.experimental.pallas import tpu as pltpu
from jax.experimental.pallas import tpu_sc as plsc

N = 10000
PADN = 10240
E = 320000
FEAT = 128
HALF = 64
GRAPHS = 64
NUM_SC = 2
NUM_SUBCORES = 16
NUM_WORKERS = NUM_SC * NUM_SUBCORES
CHUNK = 128
EDGES_PER_WORKER = E // NUM_WORKERS
CPW_PAD = 80
STAGE = 16
NBUF = 4
SPLIT = 11
ROWS_PER_SUBCORE = PADN // NUM_SUBCORES


def _sc_agg_kernel(h0_hbm, h1_hbm, src_hbm, dst_hbm, out0_hbm, out1_hbm,
                   src_v, dst_v, r0, r1, r2, r3, h_sp, accum, sem_g, sem_s):
    cid = lax.axis_index("c")
    sid = lax.axis_index("s")
    wid = cid * NUM_SUBCORES + sid
    rows = (r0, r1, r2, r3)

    zvec = jnp.zeros((16,), jnp.float32)

    for h_hbm, out_hbm in ((h0_hbm, out0_hbm), (h1_hbm, out1_hbm)):
        @pl.when(sid < NUM_SUBCORES - 1)
        def _():
            pltpu.sync_copy(h_hbm.at[pl.ds(sid * ROWS_PER_SUBCORE,
                                           ROWS_PER_SUBCORE)],
                            h_sp.at[pl.ds(sid * ROWS_PER_SUBCORE,
                                          ROWS_PER_SUBCORE)])

        @pl.when(sid == NUM_SUBCORES - 1)
        def _():
            pltpu.sync_copy(h_hbm.at[pl.ds(9600, N - 9600)],
                            h_sp.at[pl.ds(9600, N - 9600)])

        @pl.loop(0, CHUNK)
        def _(r):
            @pl.loop(0, HALF // 16)
            def _(c):
                r0[r, pl.ds(c * 16, 16)] = zvec

        @pl.loop(0, PADN // CHUNK // NUM_SUBCORES)
        def _(t):
            chunk = sid * (PADN // CHUNK // NUM_SUBCORES) + t
            pltpu.sync_copy(r0, accum.at[pl.ds(chunk * CHUNK, CHUNK)])

        plsc.subcore_barrier()

        def edge_loop(h_src):
            @pl.loop(0, CPW_PAD // STAGE)
            def _(s):
                pltpu.sync_copy(src_hbm.at[wid].at[pl.ds(s * STAGE, STAGE)],
                                src_v)
                pltpu.sync_copy(dst_hbm.at[wid].at[pl.ds(s * STAGE, STAGE)],
                                dst_v)

                for j in range(NBUF - 1):
                    pltpu.async_copy(h_src.at[src_v.at[j]], rows[j % NBUF],
                                     sem_g)

                for j in range(STAGE):
                    buf = rows[j % NBUF]
                    pltpu.make_async_copy(h_src.at[src_v.at[j]], buf,
                                          sem_g).wait()
                    pltpu.async_copy(buf, accum.at[dst_v.at[j]], sem_s,
                                     add=True)
                    if j >= 1:
                        prev = rows[(j - 1) % NBUF]
                        pltpu.make_async_copy(prev, accum.at[dst_v.at[j - 1]],
                                              sem_s).wait()
                    if j + NBUF - 1 < STAGE:
                        nxt = rows[(j + NBUF - 1) % NBUF]
                        pltpu.async_copy(h_src.at[src_v.at[j + NBUF - 1]],
                                         nxt, sem_g)

                pltpu.make_async_copy(rows[(STAGE - 1) % NBUF],
                                      accum.at[dst_v.at[STAGE - 1]],
                                      sem_s).wait()

        @pl.when(sid < SPLIT)
        def _():
            edge_loop(h_sp)

        @pl.when(sid >= SPLIT)
        def _():
            edge_loop(h_hbm)

        plsc.subcore_barrier()

        pltpu.sync_copy(accum.at[pl.ds(sid * ROWS_PER_SUBCORE,
                                       ROWS_PER_SUBCORE)],
                        out_hbm.at[cid].at[pl.ds(sid * ROWS_PER_SUBCORE,
                                                 ROWS_PER_SUBCORE)])
        plsc.subcore_barrier()


@jax.jit
def _sc_agg(h0, h1, src3d, dst3d):
    mesh = plsc.VectorSubcoreMesh(core_axis_name="c", subcore_axis_name="s")
    run = pl.kernel(
        _sc_agg_kernel,
        out_type=(
            jax.ShapeDtypeStruct((NUM_SC, PADN, HALF), jnp.float32),
            jax.ShapeDtypeStruct((NUM_SC, PADN, HALF), jnp.float32),
        ),
        mesh=mesh,
        compiler_params=pltpu.CompilerParams(use_tc_tiling_on_sc=False),
        scratch_types=[
            pltpu.VMEM((STAGE, CHUNK), jnp.int32),
            pltpu.VMEM((STAGE, CHUNK), jnp.int32),
            pltpu.VMEM((CHUNK, HALF), jnp.float32),
            pltpu.VMEM((CHUNK, HALF), jnp.float32),
            pltpu.VMEM((CHUNK, HALF), jnp.float32),
            pltpu.VMEM((CHUNK, HALF), jnp.float32),
            pltpu.VMEM_SHARED((PADN, HALF), jnp.float32),
            pltpu.VMEM_SHARED((PADN, HALF), jnp.float32),
            pltpu.SemaphoreType.DMA,
            pltpu.SemaphoreType.DMA,
        ],
    )
    return run(h0, h1, src3d, dst3d)


def _tc_layer_kernel(h0_ref, h1_ref, p0_ref, p1_ref, w1_ref, b1_ref, w2_ref,
                     b2_ref, g_ref, bt_ref, batch_ref, hout0_ref, hout1_ref,
                     pool_ref):
    h = jnp.concatenate([h0_ref[...], h1_ref[...]], axis=1)
    p0 = p0_ref[...]
    p1 = p1_ref[...]
    agg = jnp.concatenate([p0[0, :N, :] + p0[1, :N, :],
                           p1[0, :N, :] + p1[1, :N, :]], axis=1)
    h2 = h + agg
    a = jnp.maximum(
        jax.lax.dot_general(h2, w1_ref[...], (((1,), (0,)), ((), ())),
                            preferred_element_type=jnp.float32) + b1_ref[...],
        0.0)
    z = jax.lax.dot_general(a, w2_ref[...], (((1,), (0,)), ((), ())),
                            preferred_element_type=jnp.float32) + b2_ref[...]
    z = jnp.maximum(z, 0.0)
    mean = jnp.mean(z, axis=0, keepdims=True)
    var = jnp.mean((z - mean) ** 2, axis=0, keepdims=True)
    hout = g_ref[...] * (z - mean) * lax.rsqrt(var + 1e-5) + bt_ref[...]
    hout0_ref[...] = hout[:, :HALF]
    hout1_ref[...] = hout[:, HALF:]
    seg = jnp.broadcast_to(batch_ref[...], (GRAPHS, N))
    gid = lax.broadcasted_iota(jnp.int32, (GRAPHS, N), 0)
    ph = jnp.where(seg == gid, 1.0, 0.0).astype(jnp.float32)
    pool_ref[...] = jax.lax.dot_general(ph, hout, (((1,), (0,)), ((), ())),
                                        preferred_element_type=jnp.float32)


@jax.jit
def _tc_layer(h0, h1, p0, p1, w1, b1, w2, b2, gamma, beta, batch2d):
    return pl.pallas_call(
        _tc_layer_kernel,
        out_shape=(
            jax.ShapeDtypeStruct((N, HALF), jnp.float32),
            jax.ShapeDtypeStruct((N, HALF), jnp.float32),
            jax.ShapeDtypeStruct((GRAPHS, FEAT), jnp.float32),
        ),
    )(h0, h1, p0, p1, w1, b1.reshape(1, FEAT), w2, b2.reshape(1, FEAT),
      gamma.reshape(1, FEAT), beta.reshape(1, FEAT), batch2d)


def kernel(x, edge_index, batch, W1_0, b1_0, W2_0, b2_0, gamma_0, beta_0,
           W1_1, b1_1, W2_1, b2_1, gamma_1, beta_1, W1_2, b1_2, W2_2, b2_2,
           gamma_2, beta_2):
    padlen = CPW_PAD * CHUNK - EDGES_PER_WORKER
    arange = jnp.arange(padlen, dtype=jnp.int32)
    pad_src = jnp.broadcast_to((arange * 41) % N, (NUM_WORKERS, padlen))
    pad_dst = jnp.broadcast_to(N + arange % (PADN - N),
                               (NUM_WORKERS, padlen))
    src3d = jnp.concatenate(
        [edge_index[0].reshape(NUM_WORKERS, EDGES_PER_WORKER), pad_src],
        axis=1).reshape(NUM_WORKERS, CPW_PAD, CHUNK)
    dst3d = jnp.concatenate(
        [edge_index[1].reshape(NUM_WORKERS, EDGES_PER_WORKER), pad_dst],
        axis=1).reshape(NUM_WORKERS, CPW_PAD, CHUNK)
    batch2d = batch.reshape(1, N)
    params = [
        (W1_0, b1_0, W2_0, b2_0, gamma_0, beta_0),
        (W1_1, b1_1, W2_1, b2_1, gamma_1, beta_1),
        (W1_2, b1_2, W2_2, b2_2, gamma_2, beta_2),
    ]
    h0, h1 = x[:, :HALF], x[:, HALF:]
    hs, pools = [], []
    for (w1, b1, w2, b2, g, b) in params:
        p0, p1 = _sc_agg(h0, h1, src3d, dst3d)
        h0, h1, pool = _tc_layer(h0, h1, p0, p1, w1, b1, w2, b2, g, b,
                                 batch2d)
        hs.extend([h0, h1])
        pools.append(pool)
    return (jnp.concatenate(pools, axis=1), jnp.concatenate(hs, axis=1))


# speedup vs baseline: 1.1247x; 1.1247x over previous
import jax
import jax.numpy as jnp
from jax import lax
from jax.experimental import pallas as pl
from jax.experimental.pallas import tpu as pltpu
from jax.experimental.pallas import tpu_sc as plsc

N = 10000
PADN = 10240
E = 320000
FEAT = 128
HALF = 64
GRAPHS = 64
NUM_SC = 2
NUM_SUBCORES = 16
NUM_WORKERS = NUM_SC * NUM_SUBCORES
CHUNK = 128
EDGES_PER_WORKER = E // NUM_WORKERS
CPW_PAD = 80
STAGE = 16
NBUF = 4
SPLIT = 11
ROWS_PER_SUBCORE = PADN // NUM_SUBCORES


def _sc_agg_kernel(h0_hbm, h1_hbm, src_hbm, dst_hbm, out0_hbm, out1_hbm,
                   src_v, dst_v, r0, r1, r2, r3, h_sp, accum, sem_g, sem_s):
    cid = lax.axis_index("c")
    sid = lax.axis_index("s")
    wid = cid * NUM_SUBCORES + sid
    rows = (r0, r1, r2, r3)

    zvec = jnp.zeros((16,), jnp.float32)

    for h_hbm, out_hbm in ((h0_hbm, out0_hbm), (h1_hbm, out1_hbm)):
        @pl.when(sid < NUM_SUBCORES - 1)
        def _():
            pltpu.sync_copy(h_hbm.at[pl.ds(sid * ROWS_PER_SUBCORE,
                                           ROWS_PER_SUBCORE)],
                            h_sp.at[pl.ds(sid * ROWS_PER_SUBCORE,
                                          ROWS_PER_SUBCORE)])

        @pl.when(sid == NUM_SUBCORES - 1)
        def _():
            pltpu.sync_copy(h_hbm.at[pl.ds(9600, N - 9600)],
                            h_sp.at[pl.ds(9600, N - 9600)])

        @pl.loop(0, CHUNK)
        def _(r):
            @pl.loop(0, HALF // 16)
            def _(c):
                r0[r, pl.ds(c * 16, 16)] = zvec

        @pl.loop(0, PADN // CHUNK // NUM_SUBCORES)
        def _(t):
            chunk = sid * (PADN // CHUNK // NUM_SUBCORES) + t
            pltpu.sync_copy(r0, accum.at[pl.ds(chunk * CHUNK, CHUNK)])

        plsc.subcore_barrier()

        def edge_loop(h_src, s_lo, s_hi):
            @pl.loop(s_lo, s_hi)
            def _(s):
                pltpu.sync_copy(src_hbm.at[wid].at[pl.ds(s * STAGE, STAGE)],
                                src_v)
                pltpu.sync_copy(dst_hbm.at[wid].at[pl.ds(s * STAGE, STAGE)],
                                dst_v)

                for j in range(NBUF - 1):
                    pltpu.async_copy(h_src.at[src_v.at[j]], rows[j % NBUF],
                                     sem_g)

                for j in range(STAGE):
                    buf = rows[j % NBUF]
                    pltpu.make_async_copy(h_src.at[src_v.at[j]], buf,
                                          sem_g).wait()
                    pltpu.async_copy(buf, accum.at[dst_v.at[j]], sem_s,
                                     add=True)
                    if j >= 1:
                        prev = rows[(j - 1) % NBUF]
                        pltpu.make_async_copy(prev, accum.at[dst_v.at[j - 1]],
                                              sem_s).wait()
                    if j + NBUF - 1 < STAGE:
                        nxt = rows[(j + NBUF - 1) % NBUF]
                        pltpu.async_copy(h_src.at[src_v.at[j + NBUF - 1]],
                                         nxt, sem_g)

                pltpu.make_async_copy(rows[(STAGE - 1) % NBUF],
                                      accum.at[dst_v.at[STAGE - 1]],
                                      sem_s).wait()

        @pl.when(sid < SPLIT)
        def _():
            edge_loop(h_sp, 0, 3)
            edge_loop(h_hbm, 3, 5)

        @pl.when(sid >= SPLIT)
        def _():
            edge_loop(h_sp, 0, 4)
            edge_loop(h_hbm, 4, 5)

        plsc.subcore_barrier()

        pltpu.sync_copy(accum.at[pl.ds(sid * ROWS_PER_SUBCORE,
                                       ROWS_PER_SUBCORE)],
                        out_hbm.at[cid].at[pl.ds(sid * ROWS_PER_SUBCORE,
                                                 ROWS_PER_SUBCORE)])
        plsc.subcore_barrier()


@jax.jit
def _sc_agg(h0, h1, src3d, dst3d):
    mesh = plsc.VectorSubcoreMesh(core_axis_name="c", subcore_axis_name="s")
    run = pl.kernel(
        _sc_agg_kernel,
        out_type=(
            jax.ShapeDtypeStruct((NUM_SC, PADN, HALF), jnp.float32),
            jax.ShapeDtypeStruct((NUM_SC, PADN, HALF), jnp.float32),
        ),
        mesh=mesh,
        compiler_params=pltpu.CompilerParams(use_tc_tiling_on_sc=False),
        scratch_types=[
            pltpu.VMEM((STAGE, CHUNK), jnp.int32),
            pltpu.VMEM((STAGE, CHUNK), jnp.int32),
            pltpu.VMEM((CHUNK, HALF), jnp.float32),
            pltpu.VMEM((CHUNK, HALF), jnp.float32),
            pltpu.VMEM((CHUNK, HALF), jnp.float32),
            pltpu.VMEM((CHUNK, HALF), jnp.float32),
            pltpu.VMEM_SHARED((PADN, HALF), jnp.float32),
            pltpu.VMEM_SHARED((PADN, HALF), jnp.float32),
            pltpu.SemaphoreType.DMA,
            pltpu.SemaphoreType.DMA,
        ],
    )
    return run(h0, h1, src3d, dst3d)


def _tc_layer_kernel(h0_ref, h1_ref, p0_ref, p1_ref, w1_ref, b1_ref, w2_ref,
                     b2_ref, g_ref, bt_ref, batch_ref, hout0_ref, hout1_ref,
                     pool_ref):
    h = jnp.concatenate([h0_ref[...], h1_ref[...]], axis=1)
    p0 = p0_ref[...]
    p1 = p1_ref[...]
    agg = jnp.concatenate([p0[0, :N, :] + p0[1, :N, :],
                           p1[0, :N, :] + p1[1, :N, :]], axis=1)
    h2 = h + agg
    a = jnp.maximum(
        jax.lax.dot_general(h2, w1_ref[...], (((1,), (0,)), ((), ())),
                            preferred_element_type=jnp.float32) + b1_ref[...],
        0.0)
    z = jax.lax.dot_general(a, w2_ref[...], (((1,), (0,)), ((), ())),
                            preferred_element_type=jnp.float32) + b2_ref[...]
    z = jnp.maximum(z, 0.0)
    mean = jnp.mean(z, axis=0, keepdims=True)
    var = jnp.mean((z - mean) ** 2, axis=0, keepdims=True)
    hout = g_ref[...] * (z - mean) * lax.rsqrt(var + 1e-5) + bt_ref[...]
    hout0_ref[...] = hout[:, :HALF]
    hout1_ref[...] = hout[:, HALF:]
    seg = jnp.broadcast_to(batch_ref[...], (GRAPHS, N))
    gid = lax.broadcasted_iota(jnp.int32, (GRAPHS, N), 0)
    ph = jnp.where(seg == gid, 1.0, 0.0).astype(jnp.float32)
    pool_ref[...] = jax.lax.dot_general(ph, hout, (((1,), (0,)), ((), ())),
                                        preferred_element_type=jnp.float32)


@jax.jit
def _tc_layer(h0, h1, p0, p1, w1, b1, w2, b2, gamma, beta, batch2d):
    return pl.pallas_call(
        _tc_layer_kernel,
        out_shape=(
            jax.ShapeDtypeStruct((N, HALF), jnp.float32),
            jax.ShapeDtypeStruct((N, HALF), jnp.float32),
            jax.ShapeDtypeStruct((GRAPHS, FEAT), jnp.float32),
        ),
    )(h0, h1, p0, p1, w1, b1.reshape(1, FEAT), w2, b2.reshape(1, FEAT),
      gamma.reshape(1, FEAT), beta.reshape(1, FEAT), batch2d)


def kernel(x, edge_index, batch, W1_0, b1_0, W2_0, b2_0, gamma_0, beta_0,
           W1_1, b1_1, W2_1, b2_1, gamma_1, beta_1, W1_2, b1_2, W2_2, b2_2,
           gamma_2, beta_2):
    padlen = CPW_PAD * CHUNK - EDGES_PER_WORKER
    arange = jnp.arange(padlen, dtype=jnp.int32)
    pad_src = jnp.broadcast_to((arange * 41) % N, (NUM_WORKERS, padlen))
    pad_dst = jnp.broadcast_to(N + arange % (PADN - N),
                               (NUM_WORKERS, padlen))
    src3d = jnp.concatenate(
        [edge_index[0].reshape(NUM_WORKERS, EDGES_PER_WORKER), pad_src],
        axis=1).reshape(NUM_WORKERS, CPW_PAD, CHUNK)
    dst3d = jnp.concatenate(
        [edge_index[1].reshape(NUM_WORKERS, EDGES_PER_WORKER), pad_dst],
        axis=1).reshape(NUM_WORKERS, CPW_PAD, CHUNK)
    batch2d = batch.reshape(1, N)
    params = [
        (W1_0, b1_0, W2_0, b2_0, gamma_0, beta_0),
        (W1_1, b1_1, W2_1, b2_1, gamma_1, beta_1),
        (W1_2, b1_2, W2_2, b2_2, gamma_2, beta_2),
    ]
    h0, h1 = x[:, :HALF], x[:, HALF:]
    hs, pools = [], []
    for (w1, b1, w2, b2, g, b) in params:
        p0, p1 = _sc_agg(h0, h1, src3d, dst3d)
        h0, h1, pool = _tc_layer(h0, h1, p0, p1, w1, b1, w2, b2, g, b,
                                 batch2d)
        hs.extend([h0, h1])
        pools.append(pool)
    return (jnp.concatenate(pools, axis=1), jnp.concatenate(hs, axis=1))


# speedup vs baseline: 1.1287x; 1.0036x over previous
import jax
import jax.numpy as jnp
from jax import lax
from jax.experimental import pallas as pl
from jax.experimental.pallas import tpu as pltpu
from jax.experimental.pallas import tpu_sc as plsc

N = 10000
PADN = 10240
E = 320000
FEAT = 128
HALF = 64
GRAPHS = 64
NUM_SC = 2
NUM_SUBCORES = 16
NUM_WORKERS = NUM_SC * NUM_SUBCORES
CHUNK = 128
EDGES_PER_WORKER = E // NUM_WORKERS
CPW_PAD = 80
STAGE = 16
NBUF = 4
ROWS_PER_SUBCORE = PADN // NUM_SUBCORES


def _sc_agg_kernel(h0_hbm, h1_hbm, src_hbm, dst_hbm, out0_hbm, out1_hbm,
                   src_v, dst_v, r0, r1, r2, r3, h_sp, accum, sem_g, sem_s):
    cid = lax.axis_index("c")
    sid = lax.axis_index("s")
    wid = cid * NUM_SUBCORES + sid
    rows = (r0, r1, r2, r3)

    zvec = jnp.zeros((16,), jnp.float32)

    for h_hbm, out_hbm in ((h0_hbm, out0_hbm), (h1_hbm, out1_hbm)):
        @pl.when(sid < NUM_SUBCORES - 1)
        def _():
            pltpu.sync_copy(h_hbm.at[pl.ds(sid * ROWS_PER_SUBCORE,
                                           ROWS_PER_SUBCORE)],
                            h_sp.at[pl.ds(sid * ROWS_PER_SUBCORE,
                                          ROWS_PER_SUBCORE)])

        @pl.when(sid == NUM_SUBCORES - 1)
        def _():
            pltpu.sync_copy(h_hbm.at[pl.ds(9600, N - 9600)],
                            h_sp.at[pl.ds(9600, N - 9600)])

        @pl.loop(0, CHUNK)
        def _(r):
            @pl.loop(0, HALF // 16)
            def _(c):
                r0[r, pl.ds(c * 16, 16)] = zvec

        @pl.loop(0, PADN // CHUNK // NUM_SUBCORES)
        def _(t):
            chunk = sid * (PADN // CHUNK // NUM_SUBCORES) + t
            pltpu.sync_copy(r0, accum.at[pl.ds(chunk * CHUNK, CHUNK)])

        plsc.subcore_barrier()

        @pl.loop(0, CPW_PAD // STAGE)
        def _(s):
            pltpu.sync_copy(src_hbm.at[wid].at[pl.ds(s * STAGE, STAGE)],
                            src_v)
            pltpu.sync_copy(dst_hbm.at[wid].at[pl.ds(s * STAGE, STAGE)],
                            dst_v)

            for j in range(NBUF - 1):
                pltpu.async_copy(h_sp.at[src_v.at[j]], rows[j % NBUF], sem_g)

            for j in range(STAGE):
                buf = rows[j % NBUF]
                pltpu.make_async_copy(h_sp.at[src_v.at[j]], buf, sem_g).wait()
                pltpu.async_copy(buf, accum.at[dst_v.at[j]], sem_s, add=True)
                if j >= 1:
                    prev = rows[(j - 1) % NBUF]
                    pltpu.make_async_copy(prev, accum.at[dst_v.at[j - 1]],
                                          sem_s).wait()
                if j + NBUF - 1 < STAGE:
                    nxt = rows[(j + NBUF - 1) % NBUF]
                    pltpu.async_copy(h_sp.at[src_v.at[j + NBUF - 1]], nxt,
                                     sem_g)

            pltpu.make_async_copy(rows[(STAGE - 1) % NBUF],
                                  accum.at[dst_v.at[STAGE - 1]],
                                  sem_s).wait()

        plsc.subcore_barrier()

        pltpu.sync_copy(accum.at[pl.ds(sid * ROWS_PER_SUBCORE,
                                       ROWS_PER_SUBCORE)],
                        out_hbm.at[cid].at[pl.ds(sid * ROWS_PER_SUBCORE,
                                                 ROWS_PER_SUBCORE)])
        plsc.subcore_barrier()


@jax.jit
def _sc_agg(h0, h1, src3d, dst3d):
    mesh = plsc.VectorSubcoreMesh(core_axis_name="c", subcore_axis_name="s")
    run = pl.kernel(
        _sc_agg_kernel,
        out_type=(
            jax.ShapeDtypeStruct((NUM_SC, PADN, HALF), jnp.float32),
            jax.ShapeDtypeStruct((NUM_SC, PADN, HALF), jnp.float32),
        ),
        mesh=mesh,
        compiler_params=pltpu.CompilerParams(use_tc_tiling_on_sc=False),
        scratch_types=[
            pltpu.VMEM((STAGE, CHUNK), jnp.int32),
            pltpu.VMEM((STAGE, CHUNK), jnp.int32),
            pltpu.VMEM((CHUNK, HALF), jnp.float32),
            pltpu.VMEM((CHUNK, HALF), jnp.float32),
            pltpu.VMEM((CHUNK, HALF), jnp.float32),
            pltpu.VMEM((CHUNK, HALF), jnp.float32),
            pltpu.VMEM_SHARED((PADN, HALF), jnp.float32),
            pltpu.VMEM_SHARED((PADN, HALF), jnp.float32),
            pltpu.SemaphoreType.DMA,
            pltpu.SemaphoreType.DMA,
        ],
    )
    return run(h0, h1, src3d, dst3d)


def _tc_layer_kernel(h0_ref, h1_ref, p0_ref, p1_ref, w1_ref, b1_ref, w2_ref,
                     b2_ref, g_ref, bt_ref, batch_ref, hout0_ref, hout1_ref,
                     pool_ref):
    h = jnp.concatenate([h0_ref[...], h1_ref[...]], axis=1)
    p0 = p0_ref[...]
    p1 = p1_ref[...]
    agg = jnp.concatenate([p0[0, :N, :] + p0[1, :N, :],
                           p1[0, :N, :] + p1[1, :N, :]], axis=1)
    h2 = h + agg
    a = jnp.maximum(
        jax.lax.dot_general(h2, w1_ref[...], (((1,), (0,)), ((), ())),
                            preferred_element_type=jnp.float32) + b1_ref[...],
        0.0)
    z = jax.lax.dot_general(a, w2_ref[...], (((1,), (0,)), ((), ())),
                            preferred_element_type=jnp.float32) + b2_ref[...]
    z = jnp.maximum(z, 0.0)
    mean = jnp.mean(z, axis=0, keepdims=True)
    var = jnp.mean((z - mean) ** 2, axis=0, keepdims=True)
    hout = g_ref[...] * (z - mean) * lax.rsqrt(var + 1e-5) + bt_ref[...]
    hout0_ref[...] = hout[:, :HALF]
    hout1_ref[...] = hout[:, HALF:]
    seg = jnp.broadcast_to(batch_ref[...], (GRAPHS, N))
    gid = lax.broadcasted_iota(jnp.int32, (GRAPHS, N), 0)
    ph = jnp.where(seg == gid, 1.0, 0.0).astype(jnp.float32)
    pool_ref[...] = jax.lax.dot_general(ph, hout, (((1,), (0,)), ((), ())),
                                        preferred_element_type=jnp.float32)


@jax.jit
def _tc_layer(h0, h1, p0, p1, w1, b1, w2, b2, gamma, beta, batch2d):
    return pl.pallas_call(
        _tc_layer_kernel,
        out_shape=(
            jax.ShapeDtypeStruct((N, HALF), jnp.float32),
            jax.ShapeDtypeStruct((N, HALF), jnp.float32),
            jax.ShapeDtypeStruct((GRAPHS, FEAT), jnp.float32),
        ),
    )(h0, h1, p0, p1, w1, b1.reshape(1, FEAT), w2, b2.reshape(1, FEAT),
      gamma.reshape(1, FEAT), beta.reshape(1, FEAT), batch2d)


def kernel(x, edge_index, batch, W1_0, b1_0, W2_0, b2_0, gamma_0, beta_0,
           W1_1, b1_1, W2_1, b2_1, gamma_1, beta_1, W1_2, b1_2, W2_2, b2_2,
           gamma_2, beta_2):
    padlen = CPW_PAD * CHUNK - EDGES_PER_WORKER
    arange = jnp.arange(padlen, dtype=jnp.int32)
    pad_src = jnp.broadcast_to((arange * 41) % N, (NUM_WORKERS, padlen))
    pad_dst = jnp.broadcast_to(N + arange % (PADN - N),
                               (NUM_WORKERS, padlen))
    src3d = jnp.concatenate(
        [edge_index[0].reshape(NUM_WORKERS, EDGES_PER_WORKER), pad_src],
        axis=1).reshape(NUM_WORKERS, CPW_PAD, CHUNK)
    dst3d = jnp.concatenate(
        [edge_index[1].reshape(NUM_WORKERS, EDGES_PER_WORKER), pad_dst],
        axis=1).reshape(NUM_WORKERS, CPW_PAD, CHUNK)
    batch2d = batch.reshape(1, N)
    params = [
        (W1_0, b1_0, W2_0, b2_0, gamma_0, beta_0),
        (W1_1, b1_1, W2_1, b2_1, gamma_1, beta_1),
        (W1_2, b1_2, W2_2, b2_2, gamma_2, beta_2),
    ]
    h0, h1 = x[:, :HALF], x[:, HALF:]
    hs, pools = [], []
    for (w1, b1, w2, b2, g, b) in params:
        p0, p1 = _sc_agg(h0, h1, src3d, dst3d)
        h0, h1, pool = _tc_layer(h0, h1, p0, p1, w1, b1, w2, b2, g, b,
                                 batch2d)
        hs.extend([h0, h1])
        pools.append(pool)
    return (jnp.concatenate(pools, axis=1), jnp.concatenate(hs, axis=1))
